# Initial kernel scaffold; baseline (speedup 1.0000x reference)
import jax
import jax.numpy as jnp
from jax.experimental import pallas as pl


def kernel(x, edge_index, edge_attr, question_emb):
    raise NotImplementedError("write your pallas kernel here")


# speedup vs baseline: 2.1166x; 2.1166x over previous
import functools

import jax
import jax.numpy as jnp
from jax import lax
from jax.experimental import pallas as pl
from jax.experimental.pallas import tpu as pltpu
from jax.experimental.pallas import tpu_sc as plsc

_NUM_NODES = 10000
_NUM_EDGES = 320000
_D = 128
_ALPHA = 0.5
_NUM_LAYERS = 2

_NC = 2
_NS = 16
_FPT = _D // (_NC * _NS)
_C = 8000
_NCHUNK = _NUM_EDGES // _C
_GROUPS = _C // 16
_NODE_GROUPS = _NUM_NODES // 16

_SIM_B = 512


def _sim_body(ea_ref, q_ref, out_ref):
    ea = ea_ref[...]
    q = q_ref[...]
    qnorm = jnp.maximum(jnp.sqrt(jnp.sum(q * q)), 1e-12)
    dots = jnp.sum(ea * q, axis=1)
    rnorm = jnp.maximum(jnp.sqrt(jnp.sum(ea * ea, axis=1)), 1e-12)
    out_ref[...] = (dots / (rnorm * qnorm)).reshape(1, 8, _SIM_B // 8)


def _edge_sim(edge_attr, question_emb):
    grid = _NUM_EDGES // _SIM_B
    out = pl.pallas_call(
        _sim_body,
        grid=(grid,),
        in_specs=[
            pl.BlockSpec((_SIM_B, _D), lambda i: (i, 0)),
            pl.BlockSpec((1, _D), lambda i: (0, 0)),
        ],
        out_specs=pl.BlockSpec((1, 8, _SIM_B // 8), lambda i: (i, 0, 0)),
        out_shape=jax.ShapeDtypeStruct((grid, 8, _SIM_B // 8), jnp.float32),
    )(edge_attr, question_emb.reshape(1, _D))
    return out.reshape(_NUM_EDGES)


def _mpnn_body(xt_hbm, src_hbm, dst_hbm, sim_hbm, out_hbm,
               xc, acc, deg, srcb, dstb, simb, sem):
    wid = lax.axis_index("s") * _NC + lax.axis_index("c")
    base = wid * _FPT * _NUM_NODES

    foffs = [jnp.full((16,), f * _NUM_NODES, dtype=jnp.int32)
             for f in range(_FPT)]
    ones16 = jnp.ones((16,), jnp.float32)
    zeros16 = jnp.zeros((16,), jnp.float32)

    pltpu.sync_copy(xt_hbm.at[pl.ds(base, _FPT * _NUM_NODES)], xc)

    def zero_deg(j, carry):
        deg[pl.ds(j * 16, 16)] = zeros16
        return carry

    lax.fori_loop(0, _NODE_GROUPS, zero_deg, 0)

    for layer in range(_NUM_LAYERS):
        def zero_acc(j, carry):
            acc[pl.ds(j * 16, 16)] = zeros16
            return carry

        lax.fori_loop(0, _FPT * _NODE_GROUPS, zero_acc, 0)

        count_deg = layer == 0

        def group_body(g, carry):
            ds = pl.ds(g * 16, 16)
            s16 = srcb[ds]
            d16 = dstb[ds]
            w16 = simb[ds]
            for f in range(_FPT):
                v = plsc.load_gather(xc, [s16 + foffs[f]])
                plsc.addupdate_scatter(acc, [d16 + foffs[f]], v * w16)
            if count_deg:
                plsc.addupdate_scatter(deg, [d16], ones16)
            return carry

        def chunk_body(ch, carry):
            off = ch * _C
            c1 = pltpu.async_copy(src_hbm.at[pl.ds(off, _C)], srcb, sem)
            c2 = pltpu.async_copy(dst_hbm.at[pl.ds(off, _C)], dstb, sem)
            c3 = pltpu.async_copy(sim_hbm.at[pl.ds(off, _C)], simb, sem)
            c1.wait()
            c2.wait()
            c3.wait()
            lax.fori_loop(0, _GROUPS, group_body, 0)
            return carry

        lax.fori_loop(0, _NCHUNK, chunk_body, 0)

        if layer == 0:
            def inv_body(j, carry):
                ds = pl.ds(j * 16, 16)
                deg[ds] = 1.0 / jnp.maximum(deg[ds], 1.0)
                return carry

            lax.fori_loop(0, _NODE_GROUPS, inv_body, 0)

        def blend_body(j, carry):
            ds = pl.ds(j * 16, 16)
            iv = deg[ds]
            for f in range(_FPT):
                dsf = pl.ds(f * _NUM_NODES + j * 16, 16)
                xc[dsf] = _ALPHA * xc[dsf] + (1.0 - _ALPHA) * (acc[dsf] * iv)
            return carry

        lax.fori_loop(0, _NODE_GROUPS, blend_body, 0)

    pltpu.sync_copy(xc, out_hbm.at[pl.ds(base, _FPT * _NUM_NODES)])


def _mpnn_sc(xt, src, dst, sim):
    mesh = plsc.VectorSubcoreMesh(
        core_axis_name="c", subcore_axis_name="s",
        num_cores=_NC, num_subcores=_NS)
    run = pl.kernel(
        _mpnn_body,
        out_type=jax.ShapeDtypeStruct((_D * _NUM_NODES,), jnp.float32),
        mesh=mesh,
        scratch_types=[
            pltpu.VMEM((_FPT * _NUM_NODES,), jnp.float32),
            pltpu.VMEM((_FPT * _NUM_NODES,), jnp.float32),
            pltpu.VMEM((_NUM_NODES,), jnp.float32),
            pltpu.VMEM((_C,), jnp.int32),
            pltpu.VMEM((_C,), jnp.int32),
            pltpu.VMEM((_C,), jnp.float32),
            pltpu.SemaphoreType.DMA,
        ],
        compiler_params=pltpu.CompilerParams(needs_layout_passes=False),
    )
    return run(xt, src, dst, sim)


def kernel(x, edge_index, edge_attr, question_emb):
    src = edge_index[0].astype(jnp.int32)
    dst = edge_index[1].astype(jnp.int32)
    sim = _edge_sim(edge_attr, question_emb)
    xt = x.T.reshape(_D * _NUM_NODES)
    out_t = _mpnn_sc(xt, src, dst, sim)
    return out_t.reshape(_D, _NUM_NODES).T
